# baseline (speedup 1.0000x reference)
import functools

import jax
import jax.numpy as jnp
from jax import lax
from jax.experimental import pallas as pl
from jax.experimental.pallas import tpu as pltpu
from jax.experimental.pallas import tpu_sc as plsc

T = 8192
DIM = 768
HID = DIM // 2
MATCH_DIM = 128
WINDOW = 16
HALF = 8
MAX_PAIRS = 6
N_WIN = T // WINDOW
NP = N_WIN * MAX_PAIRS
K_TOP = max(0, min(T - int(0.7 * T), NP))
NEG_INF = float("-inf")

RB = 512
IB = 256
WB4 = 64

NC = 2
NS = 16
L = 16
NWK = NC * NS
WPW = N_WIN // NWK
RPW = WPW * WINDOW


def _ln(x, g, b):
    m = jnp.mean(x, axis=-1, keepdims=True)
    v = jnp.var(x, axis=-1, keepdims=True)
    return (x - m) / jnp.sqrt(v + 1e-5) * g + b


def _mm(a, b):
    return jnp.dot(a.astype(jnp.bfloat16), b.astype(jnp.bfloat16),
                   preferred_element_type=jnp.float32)


def _mx_kernel(x_ref, mask_ref, g1_ref, b1_ref, wm_ref, mx_ref):
    xn1 = _ln(x_ref[...], g1_ref[...], b1_ref[...])
    mx_ref[...] = _mm(xn1, wm_ref[...]) * mask_ref[...]


def _imp_kernel(x_ref, g2_ref, b2_ref, w1_ref, bb1_ref, w2_ref, bb2_ref,
                imp_ref):
    xn2 = _ln(x_ref[...], g2_ref[...], b2_ref[...])
    h = jnp.maximum(_mm(xn2, w1_ref[...]) + bb1_ref[...], 0.0)
    hb = h.astype(jnp.bfloat16).astype(jnp.float32)
    imp_ref[...] = (jnp.sum(hb * w2_ref[...], axis=1, keepdims=True)
                    + bb2_ref[...])


def _sc_match_body(mx_hbm, pa_hbm, pb_hbm, sv_hbm, mxv, pav, pbv, svv):
    wid = lax.axis_index("s") * NC + lax.axis_index("c")
    rbase = wid * RPW
    pltpu.sync_copy(mx_hbm.at[pl.ds(rbase, RPW)], mxv)

    lane = lax.iota(jnp.int32, L)

    def win_loop(win, carry):
        r0 = win * WINDOW
        sims = []
        for p in range(HALF * HALF):
            a = p // HALF
            b = p % HALF
            acc = jnp.zeros((L,), jnp.float32)
            for c in range(MATCH_DIM // L):
                va = mxv[r0 + 2 * a, pl.ds(c * L, L)]
                vb = mxv[r0 + 2 * b + 1, pl.ds(c * L, L)]
                acc = acc + va * vb
            tot = acc[0]
            for l in range(1, L):
                tot = tot + acc[l]
            sims.append(tot)
        used_a = jnp.int32(0)
        used_b = jnp.int32(0)
        pa_vec = jnp.zeros((L,), jnp.int32)
        pb_vec = jnp.zeros((L,), jnp.int32)
        sv_vec = jnp.zeros((L,), jnp.float32)
        for t in range(MAX_PAIRS):
            best = jnp.float32(NEG_INF)
            bidx = jnp.int32(64)
            bsim = jnp.float32(0.0)
            for p in range(HALF * HALF):
                a = p // HALF
                b = p % HALF
                sv = sims[p]
                svm = jnp.where(sv < 0.0, NEG_INF, sv)
                free = (((used_a >> a) & 1) == 0) & (((used_b >> b) & 1) == 0)
                take = free & ((svm > best) | (bidx == 64))
                best = jnp.where(take, svm, best)
                bsim = jnp.where(take, sv, bsim)
                bidx = jnp.where(take, p, bidx)
            ia = bidx // HALF
            ib = bidx - ia * HALF
            used_a = used_a | (jnp.int32(1) << ia)
            used_b = used_b | (jnp.int32(1) << ib)
            pa_vec = jnp.where(lane == t, ia, pa_vec)
            pb_vec = jnp.where(lane == t, ib, pb_vec)
            sv_vec = jnp.where(lane == t, bsim, sv_vec)
        pav[win, :] = pa_vec
        pbv[win, :] = pb_vec
        svv[win, :] = sv_vec
        return carry

    lax.fori_loop(0, WPW, win_loop, 0)
    obase = wid * WPW
    pltpu.sync_copy(pav, pa_hbm.at[pl.ds(obase, WPW)])
    pltpu.sync_copy(pbv, pb_hbm.at[pl.ds(obase, WPW)])
    pltpu.sync_copy(svv, sv_hbm.at[pl.ds(obase, WPW)])


_sc_match = functools.partial(
    pl.kernel,
    mesh=plsc.VectorSubcoreMesh(core_axis_name="c", subcore_axis_name="s"),
    out_type=[jax.ShapeDtypeStruct((N_WIN, L), jnp.int32),
              jax.ShapeDtypeStruct((N_WIN, L), jnp.int32),
              jax.ShapeDtypeStruct((N_WIN, L), jnp.float32)],
    scratch_types=[
        pltpu.VMEM((RPW, MATCH_DIM), jnp.float32),
        pltpu.VMEM((WPW, L), jnp.int32),
        pltpu.VMEM((WPW, L), jnp.int32),
        pltpu.VMEM((WPW, L), jnp.float32),
    ],
)(_sc_match_body)


def _score_kernel(pa_ref, pb_ref, sv_ref, imp_ref, out_ref):
    pa = pa_ref[...]
    pb = pb_ref[...]
    sval = sv_ref[...]
    impb = imp_ref[...]
    imp_e = impb[:, :, 0]
    imp_o = impb[:, :, 1]
    i8 = jax.lax.broadcasted_iota(jnp.int32, (N_WIN, MAX_PAIRS, HALF), 2)
    ohA = (pa[:, :, None] == i8).astype(jnp.float32)
    ohB = (pb[:, :, None] == i8).astype(jnp.float32)
    imp_i = jnp.sum(ohA * imp_e[:, None, :], axis=2)
    imp_j = jnp.sum(ohB * imp_o[:, None, :], axis=2)
    valid = sval >= 0.0
    score = sval - 0.25 * (imp_i + imp_j)
    sc = jnp.where(valid, score, NEG_INF)
    al = jax.nn.sigmoid(5.0 * (imp_i - imp_j))
    l128 = jax.lax.broadcasted_iota(jnp.int32, (N_WIN, 128), 1)
    packed = jnp.zeros((N_WIN, 128), jnp.float32)
    for t in range(MAX_PAIRS):
        packed = jnp.where(l128 == 16 + t, sc[:, t:t + 1], packed)
        packed = jnp.where(l128 == 24 + t, al[:, t:t + 1], packed)
    out_ref[...] = packed


def _rank_kernel(scol_ref, srow_ref, z_ref):
    sc = scol_ref[...]
    sr = srow_ref[...]
    gt = (sr > sc).astype(jnp.int32)
    i_glob = (jax.lax.broadcasted_iota(jnp.int32, (IB, NP), 0)
              + pl.program_id(0) * IB)
    j_iota = jax.lax.broadcasted_iota(jnp.int32, (IB, NP), 1)
    eq = ((sr == sc) & (j_iota < i_glob)).astype(jnp.int32)
    rank = jnp.sum(gt, axis=1) + jnp.sum(eq, axis=1)
    z = ((rank < K_TOP) & (sc[:, 0] > NEG_INF)).astype(jnp.float32)
    z_ref[...] = jnp.broadcast_to(z[:, None], (IB, 128))


def _apply_kernel(x_ref, pa_ref, pb_ref, z_ref, al_ref, out_ref):
    xb = x_ref[...]
    xe = xb[:, :, 0, :]
    xo = xb[:, :, 1, :]
    pa = pa_ref[...]
    pb = pb_ref[...]
    z = z_ref[...]
    al = al_ref[...]
    i8 = jax.lax.broadcasted_iota(jnp.int32, (WB4, HALF), 1)
    wE = jnp.zeros((WB4, HALF), jnp.float32)
    kill = jnp.zeros((WB4, HALF), jnp.float32)
    M = jnp.zeros((WB4, HALF, HALF), jnp.float32)
    for t in range(MAX_PAIRS):
        ohA = (i8 == pa[:, t][:, None]).astype(jnp.float32)
        ohB = (i8 == pb[:, t][:, None]).astype(jnp.float32)
        zt = z[:, t]
        wt = zt * (1.0 - al[:, t])
        M = M + (ohA * wt[:, None])[:, :, None] * ohB[:, None, :]
        wE = wE + ohA * wt[:, None]
        kill = kill + ohB * zt[:, None]
    acc = xe * (1.0 - wE)[:, :, None]
    for b in range(HALF):
        acc = acc + M[:, :, b:b + 1] * xo[:, b:b + 1, :]
    outo = xo * (1.0 - kill)[:, :, None]
    out_ref[:, :, 0:1, :] = acc[:, :, None, :]
    out_ref[:, :, 1:2, :] = outo[:, :, None, :]


def kernel(x, attn_mask, ln1_g, ln1_b, Wm, ln2_g, ln2_b, W1, b1, W2, b2):
    f32 = jnp.float32
    mask = attn_mask.astype(f32).reshape(T, 1)
    g1 = ln1_g.reshape(1, DIM)
    b1r = ln1_b.reshape(1, DIM)
    g2 = ln2_g.reshape(1, DIM)
    b2r = ln2_b.reshape(1, DIM)
    bb1 = b1.reshape(1, HID)
    bb2 = b2.reshape(1, 1)

    full = lambda shape: pl.BlockSpec(shape, lambda i: (0,) * len(shape))
    mx = pl.pallas_call(
        _mx_kernel,
        grid=(T // RB,),
        in_specs=[
            pl.BlockSpec((RB, DIM), lambda i: (i, 0)),
            pl.BlockSpec((RB, 1), lambda i: (i, 0)),
            full((1, DIM)), full((1, DIM)), full((DIM, MATCH_DIM)),
        ],
        out_specs=pl.BlockSpec((RB, MATCH_DIM), lambda i: (i, 0)),
        out_shape=jax.ShapeDtypeStruct((T, MATCH_DIM), f32),
    )(x, mask, g1, b1r, Wm)

    pa16, pb16, sval16 = _sc_match(mx)
    pa = pa16[:, :MAX_PAIRS]
    pb = pb16[:, :MAX_PAIRS]
    sval = sval16[:, :MAX_PAIRS]

    imp = pl.pallas_call(
        _imp_kernel,
        grid=(T // RB,),
        in_specs=[
            pl.BlockSpec((RB, DIM), lambda i: (i, 0)),
            full((1, DIM)), full((1, DIM)), full((DIM, HID)),
            full((1, HID)), full((1, HID)), full((1, 1)),
        ],
        out_specs=pl.BlockSpec((RB, 1), lambda i: (i, 0)),
        out_shape=jax.ShapeDtypeStruct((T, 1), f32),
    )(x, g2, b2r, W1.astype(jnp.bfloat16), bb1,
      W2.astype(jnp.bfloat16).astype(f32).reshape(1, HID), bb2)

    imp3 = imp.reshape(N_WIN, HALF, 2)
    pair_full = full((N_WIN, MAX_PAIRS))
    packed = pl.pallas_call(
        _score_kernel,
        grid=(1,),
        in_specs=[pair_full, pair_full, pair_full,
                  full((N_WIN, HALF, 2))],
        out_specs=full((N_WIN, 128)),
        out_shape=jax.ShapeDtypeStruct((N_WIN, 128), f32),
    )(pa, pb, sval, imp3)
    sc = packed[:, 16:16 + MAX_PAIRS]
    al = packed[:, 24:24 + MAX_PAIRS]

    scol = sc.reshape(NP, 1)
    srow = sc.reshape(1, NP)
    z = pl.pallas_call(
        _rank_kernel,
        grid=(NP // IB,),
        in_specs=[pl.BlockSpec((IB, 1), lambda i: (i, 0)),
                  pl.BlockSpec((1, NP), lambda i: (0, 0))],
        out_specs=pl.BlockSpec((IB, 128), lambda i: (i, 0)),
        out_shape=jax.ShapeDtypeStruct((NP, 128), f32),
    )(scol, srow)
    z2 = z[:, 0].reshape(N_WIN, MAX_PAIRS)

    x4 = x.reshape(N_WIN, HALF, 2, DIM)
    pair_spec4 = pl.BlockSpec((WB4, MAX_PAIRS), lambda i: (i, 0))
    out4 = pl.pallas_call(
        _apply_kernel,
        grid=(N_WIN // WB4,),
        in_specs=[
            pl.BlockSpec((WB4, HALF, 2, DIM), lambda i: (i, 0, 0, 0)),
            pair_spec4, pair_spec4, pair_spec4, pair_spec4,
        ],
        out_specs=pl.BlockSpec((WB4, HALF, 2, DIM), lambda i: (i, 0, 0, 0)),
        out_shape=jax.ShapeDtypeStruct((N_WIN, HALF, 2, DIM), f32),
    )(x4, pa, pb, z2, al)
    return out4.reshape(T, DIM)
